# baseline (speedup 1.0000x reference)
import functools

import jax
import jax.numpy as jnp
from jax import lax
from jax.experimental import pallas as pl
from jax.experimental.pallas import tpu as pltpu
from jax.experimental.pallas import tpu_sc as plsc

NC, NS = 2, 16
NW = NC * NS
N_PAD = 10240
W_DEG = 2000
W_PROP = 400


def _mesh():
    return plsc.VectorSubcoreMesh(
        core_axis_name="c", subcore_axis_name="s", num_cores=NC, num_subcores=NS
    )


def _params():
    return pltpu.CompilerParams(use_tc_tiling_on_sc=False)


def _deg_body(n_pad, nwin, ei_hbm, zeros_hbm, p0, p1,
              dst_v, ones_v, deg_sh, isem, ssem):
    c = lax.axis_index("c")
    s = lax.axis_index("s")
    wid = c * NS + s
    rpt = n_pad // NS
    r0 = s * rpt
    for i in range(W_DEG // 16):
        ones_v[pl.ds(i * 16, 16)] = jnp.ones((16,), jnp.float32)
    pltpu.sync_copy(zeros_hbm.at[pl.ds(r0, rpt)], deg_sh.at[pl.ds(r0, rpt)])
    plsc.subcore_barrier()

    base = wid * nwin
    idx_d = [None] * nwin
    sc_d = [None] * nwin
    for w in range(min(2, nwin)):
        idx_d[w] = pltpu.async_copy(
            ei_hbm.at[1, pl.ds((base + w) * W_DEG, W_DEG)], dst_v[w % 4], isem)
    for w in range(nwin):
        idx_d[w].wait()
        if w >= 2:
            sc_d[w - 2].wait()
        sc_d[w] = pltpu.async_copy(
            ones_v, deg_sh.at[dst_v[w % 4]], ssem, add=True)
        if w + 2 < nwin:
            idx_d[w + 2] = pltpu.async_copy(
                ei_hbm.at[1, pl.ds((base + w + 2) * W_DEG, W_DEG)],
                dst_v[(w + 2) % 4], isem)
    for w in range(max(0, nwin - 2), nwin):
        sc_d[w].wait()
    plsc.subcore_barrier()

    @pl.when(c == 0)
    def _():
        pltpu.sync_copy(deg_sh.at[pl.ds(r0, rpt)], p0.at[pl.ds(r0, rpt)])

    @pl.when(c == 1)
    def _():
        pltpu.sync_copy(deg_sh.at[pl.ds(r0, rpt)], p1.at[pl.ds(r0, rpt)])


def _deg_call(ei, zeros_col, n_pad, nwin):
    kfn = pl.kernel(
        functools.partial(_deg_body, n_pad, nwin),
        out_type=(
            jax.ShapeDtypeStruct((n_pad,), jnp.float32),
            jax.ShapeDtypeStruct((n_pad,), jnp.float32),
        ),
        mesh=_mesh(),
        scratch_types=(
            [pltpu.VMEM((W_DEG,), jnp.int32) for _ in range(4)],
            pltpu.VMEM((W_DEG,), jnp.float32),
            pltpu.VMEM_SHARED((n_pad,), jnp.float32),
            pltpu.SemaphoreType.DMA,
            pltpu.SemaphoreType.DMA,
        ),
        compiler_params=_params(),
    )
    return kfn(ei, zeros_col)


def _prop_body(n_pad, nwin, wsz, d,
               g_hbm, ei_hbm, zeros_hbm, p0, p1,
               src_v, dst_v, rows_v, acc_sh, isem, gsem, ssem):
    c = lax.axis_index("c")
    s = lax.axis_index("s")
    wid = c * NS + s
    rpt = n_pad // NS
    r0 = s * rpt
    pltpu.sync_copy(zeros_hbm.at[pl.ds(r0, rpt)], acc_sh.at[pl.ds(r0, rpt)])
    plsc.subcore_barrier()

    base = wid * nwin
    idx_d = [None] * nwin
    g_d = [None] * nwin
    sc_d = [None] * nwin

    def start_idx(w):
        idx_d[w] = (
            pltpu.async_copy(
                ei_hbm.at[0, pl.ds((base + w) * wsz, wsz)], src_v[w % 4],
                isem),
            pltpu.async_copy(
                ei_hbm.at[1, pl.ds((base + w) * wsz, wsz)], dst_v[w % 4],
                isem),
        )

    for w in range(min(2, nwin)):
        start_idx(w)
    idx_d[0][0].wait()
    idx_d[0][1].wait()
    g_d[0] = pltpu.async_copy(g_hbm.at[src_v[0]], rows_v[0], gsem)

    for w in range(nwin):
        g_d[w].wait()
        if w >= 2:
            sc_d[w - 2].wait()
        if w + 1 < nwin:
            idx_d[w + 1][0].wait()
            idx_d[w + 1][1].wait()
            g_d[w + 1] = pltpu.async_copy(
                g_hbm.at[src_v[(w + 1) % 4]], rows_v[(w + 1) % 3], gsem)
        sc_d[w] = pltpu.async_copy(
            rows_v[w % 3], acc_sh.at[dst_v[w % 4]], ssem, add=True)
        if w + 2 < nwin:
            start_idx(w + 2)
    for w in range(max(0, nwin - 2), nwin):
        sc_d[w].wait()
    plsc.subcore_barrier()

    @pl.when(c == 0)
    def _():
        pltpu.sync_copy(acc_sh.at[pl.ds(r0, rpt)], p0.at[pl.ds(r0, rpt)])

    @pl.when(c == 1)
    def _():
        pltpu.sync_copy(acc_sh.at[pl.ds(r0, rpt)], p1.at[pl.ds(r0, rpt)])


def _prop_call(g, ei, zeros_nd, n_pad, nwin, wsz, d):
    kfn = pl.kernel(
        functools.partial(_prop_body, n_pad, nwin, wsz, d),
        out_type=(
            jax.ShapeDtypeStruct((n_pad, d), jnp.float32),
            jax.ShapeDtypeStruct((n_pad, d), jnp.float32),
        ),
        mesh=_mesh(),
        scratch_types=(
            [pltpu.VMEM((wsz,), jnp.int32) for _ in range(4)],
            [pltpu.VMEM((wsz,), jnp.int32) for _ in range(4)],
            [pltpu.VMEM((wsz, d), jnp.float32) for _ in range(3)],
            pltpu.VMEM_SHARED((n_pad, d), jnp.float32),
            pltpu.SemaphoreType.DMA,
            pltpu.SemaphoreType.DMA,
            pltpu.SemaphoreType.DMA,
        ),
        compiler_params=_params(),
    )
    return kfn(g, ei, zeros_nd)


def _tc_mm1_body(x_ref, w1_ref, h1_ref):
    h1_ref[...] = jnp.dot(x_ref[...], w1_ref[...],
                          preferred_element_type=jnp.float32)


def _tc1_body(h1_ref, p0_ref, p1_ref, g1_ref, dinv_ref):
    deg = p0_ref[...] + p1_ref[...] + 1.0
    dinv = lax.rsqrt(deg)
    dinv_ref[...] = dinv
    g1_ref[...] = h1_ref[...] * dinv


def _tc2_body(n, a0_ref, a1_ref, g1_ref, dinv_ref, gamma_ref, beta_ref,
              w2_ref, g2_ref):
    dinv = dinv_ref[...]
    h = (a0_ref[pl.ds(0, n)] + a1_ref[pl.ds(0, n)] + g1_ref[...]) * dinv
    mu = jnp.mean(h, axis=0, keepdims=True)
    var = jnp.mean((h - mu) * (h - mu), axis=0, keepdims=True)
    hn = (h - mu) * lax.rsqrt(var + 1e-5) * gamma_ref[...] + beta_ref[...]
    hr = jnp.maximum(hn, 0.0)
    h2 = jnp.dot(hr, w2_ref[...], preferred_element_type=jnp.float32)
    g2_ref[...] = h2 * dinv


def _tc3_body(n, a0_ref, a1_ref, g2_ref, dinv_ref, b2_ref, out_ref):
    out_ref[...] = (a0_ref[pl.ds(0, n)] + a1_ref[pl.ds(0, n)]
                    + g2_ref[...]) * dinv_ref[...] + b2_ref[...]


def _tc_call(body, out_shapes, *args):
    return pl.pallas_call(body, out_shape=out_shapes)(*args)


def kernel(x, edge_index, W1, b1, gamma1, beta1, W2, b2):
    n = x.shape[0]
    e = edge_index.shape[1]
    d_hid = W1.shape[1]
    d_out = W2.shape[1]
    assert e % (NW * W_DEG) == 0 and e % (NW * W_PROP) == 0

    ei = edge_index.astype(jnp.int32)

    zeros_nd = jnp.zeros((N_PAD, max(d_hid, d_out)), jnp.float32)
    zeros_col = jnp.zeros((N_PAD,), jnp.float32)

    dp0, dp1 = _deg_call(ei, zeros_col, N_PAD, e // (NW * W_DEG))
    h1 = _tc_call(
        _tc_mm1_body,
        jax.ShapeDtypeStruct((n, d_hid), jnp.float32),
        x, W1,
    )
    dp0 = dp0.reshape(N_PAD, 1)[:n]
    dp1 = dp1.reshape(N_PAD, 1)[:n]

    g1, dinv = _tc_call(
        _tc1_body,
        (jax.ShapeDtypeStruct((n, d_hid), jnp.float32),
         jax.ShapeDtypeStruct((n, 1), jnp.float32)),
        h1, dp0, dp1,
    )

    a0, a1 = _prop_call(g1, ei, zeros_nd[:, :d_hid],
                        N_PAD, e // (NW * W_PROP), W_PROP, d_hid)

    g2 = _tc_call(
        functools.partial(_tc2_body, n),
        jax.ShapeDtypeStruct((n, d_out), jnp.float32),
        a0, a1, g1, dinv,
        gamma1.reshape(1, d_hid), beta1.reshape(1, d_hid), W2,
    )

    b0, b1_ = _prop_call(g2, ei, zeros_nd[:, :d_out],
                         N_PAD, e // (NW * W_PROP), W_PROP, d_out)

    out = _tc_call(
        functools.partial(_tc3_body, n),
        jax.ShapeDtypeStruct((n, d_out), jnp.float32),
        b0, b1_, g2, dinv, b2.reshape(1, d_out),
    )
    return out


# speedup vs baseline: 1.0606x; 1.0606x over previous
import functools

import jax
import jax.numpy as jnp
from jax import lax
from jax.experimental import pallas as pl
from jax.experimental.pallas import tpu as pltpu
from jax.experimental.pallas import tpu_sc as plsc

NC, NS = 2, 16
NW = NC * NS
N_PAD = 10240
W_DEG = 2000
W_PROP = 400


def _mesh():
    return plsc.VectorSubcoreMesh(
        core_axis_name="c", subcore_axis_name="s", num_cores=NC, num_subcores=NS
    )


def _params():
    return pltpu.CompilerParams(use_tc_tiling_on_sc=False)


def _deg_body(n_pad, nwin, dst_hbm, zeros_hbm, p0, p1,
              dst_v, ones_v, deg_sh, isem, ssem):
    c = lax.axis_index("c")
    s = lax.axis_index("s")
    wid = c * NS + s
    rpt = n_pad // NS
    r0 = s * rpt
    for i in range(W_DEG // 16):
        ones_v[pl.ds(i * 16, 16)] = jnp.ones((16,), jnp.float32)
    pltpu.sync_copy(zeros_hbm.at[pl.ds(r0, rpt)], deg_sh.at[pl.ds(r0, rpt)])
    plsc.subcore_barrier()

    base = wid * nwin
    idx_d = [None] * nwin
    sc_d = [None] * nwin
    for w in range(min(2, nwin)):
        idx_d[w] = pltpu.async_copy(
            dst_hbm.at[pl.ds((base + w) * W_DEG, W_DEG)], dst_v[w % 4], isem)
    for w in range(nwin):
        idx_d[w].wait()
        if w >= 2:
            sc_d[w - 2].wait()
        sc_d[w] = pltpu.async_copy(
            ones_v, deg_sh.at[dst_v[w % 4]], ssem, add=True)
        if w + 2 < nwin:
            idx_d[w + 2] = pltpu.async_copy(
                dst_hbm.at[pl.ds((base + w + 2) * W_DEG, W_DEG)],
                dst_v[(w + 2) % 4], isem)
    for w in range(max(0, nwin - 2), nwin):
        sc_d[w].wait()
    plsc.subcore_barrier()

    @pl.when(c == 0)
    def _():
        pltpu.sync_copy(deg_sh.at[pl.ds(r0, rpt)], p0.at[pl.ds(r0, rpt)])

    @pl.when(c == 1)
    def _():
        pltpu.sync_copy(deg_sh.at[pl.ds(r0, rpt)], p1.at[pl.ds(r0, rpt)])


def _deg_call(dst, zeros_col, n_pad, nwin):
    kfn = pl.kernel(
        functools.partial(_deg_body, n_pad, nwin),
        out_type=(
            jax.ShapeDtypeStruct((n_pad,), jnp.float32),
            jax.ShapeDtypeStruct((n_pad,), jnp.float32),
        ),
        mesh=_mesh(),
        scratch_types=(
            [pltpu.VMEM((W_DEG,), jnp.int32) for _ in range(4)],
            pltpu.VMEM((W_DEG,), jnp.float32),
            pltpu.VMEM_SHARED((n_pad,), jnp.float32),
            pltpu.SemaphoreType.DMA,
            pltpu.SemaphoreType.DMA,
        ),
        compiler_params=_params(),
    )
    return kfn(dst, zeros_col)


def _prop_body(n_pad, nwin, wsz, d, double_idx,
               g_hbm, src_hbm, dst_hbm, zeros_hbm, p0, p1,
               src_v, dst_v, rows_v, acc_sh, isem, gsem, ssem):
    c = lax.axis_index("c")
    s = lax.axis_index("s")
    wid = c * NS + s
    rpt = n_pad // NS
    r0 = s * rpt
    pltpu.sync_copy(zeros_hbm.at[pl.ds(r0, rpt)], acc_sh.at[pl.ds(r0, rpt)])
    plsc.subcore_barrier()

    base = wid * nwin
    idx_d = [None] * nwin
    g_d = [None] * nwin
    sc_d = [None] * nwin

    def start_idx(w):
        idx_d[w] = (
            pltpu.async_copy(
                src_hbm.at[pl.ds((base + w) * wsz, wsz)], src_v[w % 4], isem),
            pltpu.async_copy(
                dst_hbm.at[pl.ds((base + w) * wsz, wsz)], dst_v[w % 4], isem),
        )

    def wait_idx(w):
        idx_d[w][0].wait()
        idx_d[w][1].wait()
        if double_idx:
            sv = src_v[w % 4]
            for i in range(wsz // 16):
                sv[pl.ds(i * 16, 16)] = sv[pl.ds(i * 16, 16)] * 2

    for w in range(min(2, nwin)):
        start_idx(w)
    wait_idx(0)
    g_d[0] = pltpu.async_copy(g_hbm.at[src_v[0]], rows_v[0], gsem)

    for w in range(nwin):
        g_d[w].wait()
        if w >= 2:
            sc_d[w - 2].wait()
        if w + 1 < nwin:
            wait_idx(w + 1)
            g_d[w + 1] = pltpu.async_copy(
                g_hbm.at[src_v[(w + 1) % 4]], rows_v[(w + 1) % 3], gsem)
        sc_d[w] = pltpu.async_copy(
            rows_v[w % 3], acc_sh.at[dst_v[w % 4]], ssem, add=True)
        if w + 2 < nwin:
            start_idx(w + 2)
    for w in range(max(0, nwin - 2), nwin):
        sc_d[w].wait()
    plsc.subcore_barrier()

    @pl.when(c == 0)
    def _():
        pltpu.sync_copy(acc_sh.at[pl.ds(r0, rpt)], p0.at[pl.ds(r0, rpt)])

    @pl.when(c == 1)
    def _():
        pltpu.sync_copy(acc_sh.at[pl.ds(r0, rpt)], p1.at[pl.ds(r0, rpt)])


def _prop_call(g, src, dst, zeros_nd, n_pad, nwin, wsz, d, double_idx):
    kfn = pl.kernel(
        functools.partial(_prop_body, n_pad, nwin, wsz, d, double_idx),
        out_type=(
            jax.ShapeDtypeStruct((n_pad, d), jnp.float32),
            jax.ShapeDtypeStruct((n_pad, d), jnp.float32),
        ),
        mesh=_mesh(),
        scratch_types=(
            [pltpu.VMEM((wsz,), jnp.int32) for _ in range(4)],
            [pltpu.VMEM((wsz,), jnp.int32) for _ in range(4)],
            [pltpu.VMEM((wsz, d), jnp.float32) for _ in range(3)],
            pltpu.VMEM_SHARED((n_pad, d), jnp.float32),
            pltpu.SemaphoreType.DMA,
            pltpu.SemaphoreType.DMA,
            pltpu.SemaphoreType.DMA,
        ),
        compiler_params=_params(),
    )
    return kfn(g, src, dst, zeros_nd)


def _dinv_col(p0_ref, p1_ref, n):
    deg = p0_ref[pl.ds(0, n)] + p1_ref[pl.ds(0, n)] + 1.0
    return lax.rsqrt(deg).reshape(n, 1)


def _tc1_body(n, x_ref, w1a_ref, p0_ref, p1_ref, g1_ref):
    dinv = _dinv_col(p0_ref, p1_ref, n)
    h1 = jnp.dot(x_ref[...], w1a_ref[...], preferred_element_type=jnp.float32)
    g1_ref[...] = h1 * dinv


def _tc2_body(n, a0_ref, a1_ref, g1_ref, p0_ref, p1_ref, gamma_ref, beta_ref,
              w2_ref, g2_ref):
    dinv = _dinv_col(p0_ref, p1_ref, n)
    h = (a0_ref[pl.ds(0, n)] + a1_ref[pl.ds(0, n)]
         + g1_ref[pl.ds(0, n), pl.ds(0, a0_ref.shape[1])]) * dinv
    mu = jnp.mean(h, axis=0, keepdims=True)
    var = jnp.mean((h - mu) * (h - mu), axis=0, keepdims=True)
    hn = (h - mu) * lax.rsqrt(var + 1e-5) * gamma_ref[...] + beta_ref[...]
    hr = jnp.maximum(hn, 0.0)
    h2 = jnp.dot(hr, w2_ref[...], preferred_element_type=jnp.float32)
    g2_ref[...] = h2 * dinv


def _tc3_body(n, a0_ref, a1_ref, g2_ref, p0_ref, p1_ref, b2_ref, out_ref):
    dinv = _dinv_col(p0_ref, p1_ref, n)
    out_ref[...] = (a0_ref[pl.ds(0, n)] + a1_ref[pl.ds(0, n)]
                    + g2_ref[...]) * dinv + b2_ref[...]


def _tc_call(body, out_shapes, *args):
    return pl.pallas_call(body, out_shape=out_shapes)(*args)


def kernel(x, edge_index, W1, b1, gamma1, beta1, W2, b2):
    n = x.shape[0]
    e = edge_index.shape[1]
    d_hid = W1.shape[1]
    d_out = W2.shape[1]
    assert e % (NW * W_DEG) == 0 and e % (NW * W_PROP) == 0

    ei = edge_index.astype(jnp.int32)
    src = ei[0]
    dst = ei[1]

    zeros_nd = jnp.zeros((N_PAD, max(d_hid, d_out)), jnp.float32)
    zeros_col = jnp.zeros((N_PAD,), jnp.float32)

    dp0, dp1 = _deg_call(dst, zeros_col, N_PAD, e // (NW * W_DEG))

    w1_aug = jnp.concatenate([W1, W1], axis=1)
    g1 = _tc_call(
        functools.partial(_tc1_body, n),
        jax.ShapeDtypeStruct((n, 2 * d_hid), jnp.float32),
        x, w1_aug, dp0, dp1,
    )

    g1_view = g1.reshape(2 * n, d_hid)
    a0, a1 = _prop_call(g1_view, src, dst, zeros_nd[:, :d_hid],
                        N_PAD, e // (NW * W_PROP), W_PROP, d_hid, True)

    g2 = _tc_call(
        functools.partial(_tc2_body, n),
        jax.ShapeDtypeStruct((n, d_out), jnp.float32),
        a0, a1, g1, dp0, dp1,
        gamma1.reshape(1, d_hid), beta1.reshape(1, d_hid), W2,
    )

    b0, b1_ = _prop_call(g2, src, dst, zeros_nd[:, :d_out],
                         N_PAD, e // (NW * W_PROP), W_PROP, d_out, False)

    out = _tc_call(
        functools.partial(_tc3_body, n),
        jax.ShapeDtypeStruct((n, d_out), jnp.float32),
        b0, b1_, g2, dp0, dp1, b2.reshape(1, d_out),
    )
    return out


# speedup vs baseline: 1.0884x; 1.0262x over previous
import functools

import jax
import jax.numpy as jnp
from jax import lax
from jax.experimental import pallas as pl
from jax.experimental.pallas import tpu as pltpu
from jax.experimental.pallas import tpu_sc as plsc

NC, NS = 2, 16
NW = NC * NS
N_PAD = 10240
W_DEG = 2000
W_PROP = 400


def _mesh():
    return plsc.VectorSubcoreMesh(
        core_axis_name="c", subcore_axis_name="s", num_cores=NC, num_subcores=NS
    )


def _params():
    return pltpu.CompilerParams(use_tc_tiling_on_sc=False)


def _deg_body(n_pad, nwin, ei_hbm, zeros_hbm, p0, p1,
              dst_v, ones_v, deg_sh, isem, ssem):
    c = lax.axis_index("c")
    s = lax.axis_index("s")
    wid = c * NS + s
    rpt = n_pad // NS
    r0 = s * rpt
    for i in range(W_DEG // 16):
        ones_v[pl.ds(i * 16, 16)] = jnp.ones((16,), jnp.float32)
    pltpu.sync_copy(zeros_hbm.at[pl.ds(r0, rpt)], deg_sh.at[pl.ds(r0, rpt)])
    plsc.subcore_barrier()

    base = wid * nwin
    idx_d = [None] * nwin
    sc_d = [None] * nwin
    for w in range(min(2, nwin)):
        idx_d[w] = pltpu.async_copy(
            ei_hbm.at[1, pl.ds((base + w) * W_DEG, W_DEG)], dst_v[w % 4],
            isem)
    for w in range(nwin):
        idx_d[w].wait()
        if w >= 2:
            sc_d[w - 2].wait()
        sc_d[w] = pltpu.async_copy(
            ones_v, deg_sh.at[dst_v[w % 4]], ssem, add=True)
        if w + 2 < nwin:
            idx_d[w + 2] = pltpu.async_copy(
                ei_hbm.at[1, pl.ds((base + w + 2) * W_DEG, W_DEG)],
                dst_v[(w + 2) % 4], isem)
    for w in range(max(0, nwin - 2), nwin):
        sc_d[w].wait()
    plsc.subcore_barrier()

    @pl.when(c == 0)
    def _():
        pltpu.sync_copy(deg_sh.at[pl.ds(r0, rpt)], p0.at[pl.ds(r0, rpt)])

    @pl.when(c == 1)
    def _():
        pltpu.sync_copy(deg_sh.at[pl.ds(r0, rpt)], p1.at[pl.ds(r0, rpt)])


def _deg_call(ei, zeros_col, n_pad, nwin):
    kfn = pl.kernel(
        functools.partial(_deg_body, n_pad, nwin),
        out_type=(
            jax.ShapeDtypeStruct((n_pad,), jnp.float32),
            jax.ShapeDtypeStruct((n_pad,), jnp.float32),
        ),
        mesh=_mesh(),
        scratch_types=(
            [pltpu.VMEM((W_DEG,), jnp.int32) for _ in range(4)],
            pltpu.VMEM((W_DEG,), jnp.float32),
            pltpu.VMEM_SHARED((n_pad,), jnp.float32),
            pltpu.SemaphoreType.DMA,
            pltpu.SemaphoreType.DMA,
        ),
        compiler_params=_params(),
    )
    return kfn(ei, zeros_col)


def _prop_body(n_pad, nwin, wsz, d, double_idx,
               g_hbm, ei_hbm, zeros_hbm, p0, p1,
               src_v, dst_v, rows_v, acc_sh, isem, gsem, ssem):
    c = lax.axis_index("c")
    s = lax.axis_index("s")
    wid = c * NS + s
    rpt = n_pad // NS
    r0 = s * rpt
    pltpu.sync_copy(zeros_hbm.at[pl.ds(r0, rpt)], acc_sh.at[pl.ds(r0, rpt)])
    plsc.subcore_barrier()

    base = wid * nwin
    idx_d = [None] * nwin
    g_d = [None] * nwin
    sc_d = [None] * nwin

    def start_idx(w):
        idx_d[w] = (
            pltpu.async_copy(
                ei_hbm.at[0, pl.ds((base + w) * wsz, wsz)], src_v[w % 4],
                isem),
            pltpu.async_copy(
                ei_hbm.at[1, pl.ds((base + w) * wsz, wsz)], dst_v[w % 4],
                isem),
        )

    def wait_idx(w):
        idx_d[w][0].wait()
        idx_d[w][1].wait()
        if double_idx:
            sv = src_v[w % 4]
            for i in range(wsz // 16):
                sv[pl.ds(i * 16, 16)] = sv[pl.ds(i * 16, 16)] * 2

    for w in range(min(2, nwin)):
        start_idx(w)
    wait_idx(0)
    g_d[0] = pltpu.async_copy(g_hbm.at[src_v[0]], rows_v[0], gsem)

    for w in range(nwin):
        g_d[w].wait()
        if w >= 2:
            sc_d[w - 2].wait()
        if w + 1 < nwin:
            wait_idx(w + 1)
            g_d[w + 1] = pltpu.async_copy(
                g_hbm.at[src_v[(w + 1) % 4]], rows_v[(w + 1) % 3], gsem)
        sc_d[w] = pltpu.async_copy(
            rows_v[w % 3], acc_sh.at[dst_v[w % 4]], ssem, add=True)
        if w + 2 < nwin:
            start_idx(w + 2)
    for w in range(max(0, nwin - 2), nwin):
        sc_d[w].wait()
    plsc.subcore_barrier()

    @pl.when(c == 0)
    def _():
        pltpu.sync_copy(acc_sh.at[pl.ds(r0, rpt)], p0.at[pl.ds(r0, rpt)])

    @pl.when(c == 1)
    def _():
        pltpu.sync_copy(acc_sh.at[pl.ds(r0, rpt)], p1.at[pl.ds(r0, rpt)])


def _prop_call(g, ei, zeros_nd, n_pad, nwin, wsz, d, double_idx):
    kfn = pl.kernel(
        functools.partial(_prop_body, n_pad, nwin, wsz, d, double_idx),
        out_type=(
            jax.ShapeDtypeStruct((n_pad, d), jnp.float32),
            jax.ShapeDtypeStruct((n_pad, d), jnp.float32),
        ),
        mesh=_mesh(),
        scratch_types=(
            [pltpu.VMEM((wsz,), jnp.int32) for _ in range(4)],
            [pltpu.VMEM((wsz,), jnp.int32) for _ in range(4)],
            [pltpu.VMEM((wsz, d), jnp.float32) for _ in range(3)],
            pltpu.VMEM_SHARED((n_pad, d), jnp.float32),
            pltpu.SemaphoreType.DMA,
            pltpu.SemaphoreType.DMA,
            pltpu.SemaphoreType.DMA,
        ),
        compiler_params=_params(),
    )
    return kfn(g, ei, zeros_nd)


def _dinv_col(p0_ref, p1_ref, n):
    deg = p0_ref[pl.ds(0, n)] + p1_ref[pl.ds(0, n)] + 1.0
    return lax.rsqrt(deg).reshape(n, 1)


def _tc1_body(n, x_ref, w1a_ref, p0_ref, p1_ref, g1_ref):
    dinv = _dinv_col(p0_ref, p1_ref, n)
    h1 = jnp.dot(x_ref[...], w1a_ref[...], preferred_element_type=jnp.float32)
    g1_ref[...] = h1 * dinv


def _tc2_body(n, acc_ref, g1_ref, p0_ref, p1_ref, gamma_ref, beta_ref,
              w2_ref, g2_ref):
    dinv = _dinv_col(p0_ref, p1_ref, n)
    h = (acc_ref[...]
         + g1_ref[pl.ds(0, n), pl.ds(0, acc_ref.shape[1])]) * dinv
    mu = jnp.mean(h, axis=0, keepdims=True)
    var = jnp.mean((h - mu) * (h - mu), axis=0, keepdims=True)
    hn = (h - mu) * lax.rsqrt(var + 1e-5) * gamma_ref[...] + beta_ref[...]
    hr = jnp.maximum(hn, 0.0)
    h2 = jnp.dot(hr, w2_ref[...], preferred_element_type=jnp.float32)
    g2_ref[...] = h2 * dinv


def _tc3_body(n, acc_ref, g2_ref, p0_ref, p1_ref, b2_ref, out_ref):
    dinv = _dinv_col(p0_ref, p1_ref, n)
    out_ref[...] = (acc_ref[...] + g2_ref[...]) * dinv + b2_ref[...]


def _tc_call(body, out_shapes, *args):
    return pl.pallas_call(body, out_shape=out_shapes)(*args)


def kernel(x, edge_index, W1, b1, gamma1, beta1, W2, b2):
    n = x.shape[0]
    e = edge_index.shape[1]
    d_hid = W1.shape[1]
    d_out = W2.shape[1]
    assert e % (NW * W_DEG) == 0 and e % (NW * W_PROP) == 0

    ei = edge_index.astype(jnp.int32)

    zeros_nd = jnp.zeros((N_PAD, max(d_hid, d_out)), jnp.float32)
    zeros_col = jnp.zeros((N_PAD,), jnp.float32)

    dp0, dp1 = _deg_call(ei, zeros_col, N_PAD, e // (NW * W_DEG))

    w1_aug = jnp.concatenate([W1, W1], axis=1)
    g1 = _tc_call(
        functools.partial(_tc1_body, n),
        jax.ShapeDtypeStruct((n, 2 * d_hid), jnp.float32),
        x, w1_aug, dp0, dp1,
    )

    g1_view = g1.reshape(2 * n, d_hid)
    a0, a1 = _prop_call(g1_view, ei, zeros_nd[:, :d_hid],
                        N_PAD, e // (NW * W_PROP), W_PROP, d_hid, True)

    acc1 = a0[:n] + a1[:n]
    g2 = _tc_call(
        functools.partial(_tc2_body, n),
        jax.ShapeDtypeStruct((n, d_out), jnp.float32),
        acc1, g1, dp0, dp1,
        gamma1.reshape(1, d_hid), beta1.reshape(1, d_hid), W2,
    )

    b0, b1_ = _prop_call(g2, ei, zeros_nd[:, :d_out],
                         N_PAD, e // (NW * W_PROP), W_PROP, d_out, False)

    acc2 = b0[:n] + b1_[:n]
    out = _tc_call(
        functools.partial(_tc3_body, n),
        jax.ShapeDtypeStruct((n, d_out), jnp.float32),
        acc2, g2, dp0, dp1, b2.reshape(1, d_out),
    )
    return out


# speedup vs baseline: 1.1463x; 1.0532x over previous
import functools

import jax
import jax.numpy as jnp
from jax import lax
from jax.experimental import pallas as pl
from jax.experimental.pallas import tpu as pltpu
from jax.experimental.pallas import tpu_sc as plsc

NC, NS = 2, 16
NW = NC * NS
N_PAD = 10240
W_DEG = 2000
W_PROP = 400


def _mesh():
    return plsc.VectorSubcoreMesh(
        core_axis_name="c", subcore_axis_name="s", num_cores=NC, num_subcores=NS
    )


def _params():
    return pltpu.CompilerParams(use_tc_tiling_on_sc=False)


def _deg_body(n_pad, nwin, ei_hbm, zeros_hbm, p0, p1,
              dst_v, ones_v, deg_sh, isem, ssem):
    c = lax.axis_index("c")
    s = lax.axis_index("s")
    wid = c * NS + s
    rpt = n_pad // NS
    r0 = s * rpt
    for i in range(W_DEG // 16):
        ones_v[pl.ds(i * 16, 16)] = jnp.ones((16,), jnp.float32)
    pltpu.sync_copy(zeros_hbm.at[pl.ds(r0, rpt)], deg_sh.at[pl.ds(r0, rpt)])
    plsc.subcore_barrier()

    base = wid * nwin
    idx_d = [None] * nwin
    sc_d = [None] * nwin
    for w in range(min(2, nwin)):
        idx_d[w] = pltpu.async_copy(
            ei_hbm.at[1, pl.ds((base + w) * W_DEG, W_DEG)], dst_v[w % 4],
            isem)
    for w in range(nwin):
        idx_d[w].wait()
        if w >= 2:
            sc_d[w - 2].wait()
        sc_d[w] = pltpu.async_copy(
            ones_v, deg_sh.at[dst_v[w % 4]], ssem, add=True)
        if w + 2 < nwin:
            idx_d[w + 2] = pltpu.async_copy(
                ei_hbm.at[1, pl.ds((base + w + 2) * W_DEG, W_DEG)],
                dst_v[(w + 2) % 4], isem)
    for w in range(max(0, nwin - 2), nwin):
        sc_d[w].wait()
    plsc.subcore_barrier()

    @pl.when(c == 0)
    def _():
        pltpu.sync_copy(deg_sh.at[pl.ds(r0, rpt)], p0.at[pl.ds(r0, rpt)])

    @pl.when(c == 1)
    def _():
        pltpu.sync_copy(deg_sh.at[pl.ds(r0, rpt)], p1.at[pl.ds(r0, rpt)])


def _deg_call(ei, zeros_col, n_pad, nwin):
    kfn = pl.kernel(
        functools.partial(_deg_body, n_pad, nwin),
        out_type=(
            jax.ShapeDtypeStruct((n_pad,), jnp.float32),
            jax.ShapeDtypeStruct((n_pad,), jnp.float32),
        ),
        mesh=_mesh(),
        scratch_types=(
            [pltpu.VMEM((W_DEG,), jnp.int32) for _ in range(4)],
            pltpu.VMEM((W_DEG,), jnp.float32),
            pltpu.VMEM_SHARED((n_pad,), jnp.float32),
            pltpu.SemaphoreType.DMA,
            pltpu.SemaphoreType.DMA,
        ),
        compiler_params=_params(),
    )
    return kfn(ei, zeros_col)


def _prop_body(n_pad, ept, nwin, wsz, wtail, d, double_idx,
               g_hbm, ei_hbm, zeros_hbm, p0, p1,
               src_v, dst_v, rows_v, tsrc_v, tdst_v, acc_sh,
               isem, gsem, ssem):
    c = lax.axis_index("c")
    s = lax.axis_index("s")
    wid = c * NS + s
    rpt = n_pad // NS
    r0 = s * rpt
    pltpu.sync_copy(zeros_hbm.at[pl.ds(r0, rpt)], acc_sh.at[pl.ds(r0, rpt)])
    plsc.subcore_barrier()

    base = wid * ept
    idx_d = [None] * nwin
    g_d = [None] * nwin
    sc_d = [None] * nwin

    def _double(sv, m):
        for i in range(m // 16):
            sv[pl.ds(i * 16, 16)] = sv[pl.ds(i * 16, 16)] * 2

    def start_idx(w):
        off = base + w * wsz
        idx_d[w] = (
            pltpu.async_copy(ei_hbm.at[0, pl.ds(off, wsz)], src_v[w % 3],
                             isem),
            pltpu.async_copy(ei_hbm.at[1, pl.ds(off, wsz)], dst_v[w % 3],
                             isem),
        )

    def wait_idx(w):
        idx_d[w][0].wait()
        idx_d[w][1].wait()
        if double_idx:
            _double(src_v[w % 3], wsz)

    for w in range(min(2, nwin)):
        start_idx(w)
    if wtail:
        toff = base + nwin * wsz
        t_d = (pltpu.async_copy(ei_hbm.at[0, pl.ds(toff, wtail)], tsrc_v,
                                isem),
               pltpu.async_copy(ei_hbm.at[1, pl.ds(toff, wtail)], tdst_v,
                                isem))
    wait_idx(0)
    g_d[0] = pltpu.async_copy(g_hbm.at[src_v[0]], rows_v[0], gsem)

    for w in range(nwin):
        g_d[w].wait()
        if w >= 1:
            sc_d[w - 1].wait()
        if w + 1 < nwin:
            wait_idx(w + 1)
            g_d[w + 1] = pltpu.async_copy(
                g_hbm.at[src_v[(w + 1) % 3]], rows_v[(w + 1) % 2], gsem)
        sc_d[w] = pltpu.async_copy(
            rows_v[w % 2], acc_sh.at[dst_v[w % 3]], ssem, add=True)
        if w + 2 < nwin:
            start_idx(w + 2)
    sc_d[nwin - 1].wait()

    if wtail:
        t_d[0].wait()
        t_d[1].wait()
        if double_idx:
            _double(tsrc_v, wtail)
        trows = rows_v[0].at[pl.ds(0, wtail)]
        pltpu.async_copy(g_hbm.at[tsrc_v], trows, gsem).wait()
        pltpu.async_copy(trows, acc_sh.at[tdst_v], ssem, add=True).wait()
    plsc.subcore_barrier()

    @pl.when(c == 0)
    def _():
        pltpu.sync_copy(acc_sh.at[pl.ds(r0, rpt)], p0.at[pl.ds(r0, rpt)])

    @pl.when(c == 1)
    def _():
        pltpu.sync_copy(acc_sh.at[pl.ds(r0, rpt)], p1.at[pl.ds(r0, rpt)])


def _prop_call(g, ei, zeros_nd, n_pad, ept, nwin, wsz, wtail, d, double_idx):
    kfn = pl.kernel(
        functools.partial(_prop_body, n_pad, ept, nwin, wsz, wtail, d,
                          double_idx),
        out_type=(
            jax.ShapeDtypeStruct((n_pad, d), jnp.float32),
            jax.ShapeDtypeStruct((n_pad, d), jnp.float32),
        ),
        mesh=_mesh(),
        scratch_types=(
            [pltpu.VMEM((wsz,), jnp.int32) for _ in range(3)],
            [pltpu.VMEM((wsz,), jnp.int32) for _ in range(3)],
            [pltpu.VMEM((wsz, d), jnp.float32) for _ in range(2)],
            pltpu.VMEM((max(wtail, 8),), jnp.int32),
            pltpu.VMEM((max(wtail, 8),), jnp.int32),
            pltpu.VMEM_SHARED((n_pad, d), jnp.float32),
            pltpu.SemaphoreType.DMA,
            pltpu.SemaphoreType.DMA,
            pltpu.SemaphoreType.DMA,
        ),
        compiler_params=_params(),
    )
    return kfn(g, ei, zeros_nd)


def _dinv_col(p0_ref, p1_ref, n):
    deg = p0_ref[pl.ds(0, n)] + p1_ref[pl.ds(0, n)] + 1.0
    return lax.rsqrt(deg).reshape(n, 1)


def _tc1_body(n, x_ref, w1a_ref, p0_ref, p1_ref, g1_ref):
    dinv = _dinv_col(p0_ref, p1_ref, n)
    h1 = jnp.dot(x_ref[...], w1a_ref[...], preferred_element_type=jnp.float32)
    g1_ref[...] = h1 * dinv


def _tc2_body(n, acc_ref, g1_ref, p0_ref, p1_ref, gamma_ref, beta_ref,
              w2_ref, g2_ref):
    dinv = _dinv_col(p0_ref, p1_ref, n)
    h = (acc_ref[...]
         + g1_ref[pl.ds(0, n), pl.ds(0, acc_ref.shape[1])]) * dinv
    mu = jnp.mean(h, axis=0, keepdims=True)
    var = jnp.mean((h - mu) * (h - mu), axis=0, keepdims=True)
    hn = (h - mu) * lax.rsqrt(var + 1e-5) * gamma_ref[...] + beta_ref[...]
    hr = jnp.maximum(hn, 0.0)
    h2 = jnp.dot(hr, w2_ref[...], preferred_element_type=jnp.float32)
    g2_ref[...] = h2 * dinv


def _tc3_body(n, acc_ref, g2_ref, p0_ref, p1_ref, b2_ref, out_ref):
    dinv = _dinv_col(p0_ref, p1_ref, n)
    out_ref[...] = (acc_ref[...] + g2_ref[...]) * dinv + b2_ref[...]


def _tc_call(body, out_shapes, *args):
    return pl.pallas_call(body, out_shape=out_shapes)(*args)


def kernel(x, edge_index, W1, b1, gamma1, beta1, W2, b2):
    n = x.shape[0]
    e = edge_index.shape[1]
    d_hid = W1.shape[1]
    d_out = W2.shape[1]
    assert e % (NW * W_DEG) == 0 and e % (NW * W_PROP) == 0

    ei = edge_index.astype(jnp.int32)

    zeros_nd = jnp.zeros((N_PAD, max(d_hid, d_out)), jnp.float32)
    zeros_col = jnp.zeros((N_PAD,), jnp.float32)

    dp0, dp1 = _deg_call(ei, zeros_col, N_PAD, e // (NW * W_DEG))

    w1_aug = jnp.concatenate([W1, W1], axis=1)
    g1 = _tc_call(
        functools.partial(_tc1_body, n),
        jax.ShapeDtypeStruct((n, 2 * d_hid), jnp.float32),
        x, w1_aug, dp0, dp1,
    )

    ept = e // NW
    g1_view = g1.reshape(2 * n, d_hid)
    a0, a1 = _prop_call(g1_view, ei, zeros_nd[:, :d_hid],
                        N_PAD, ept, ept // 600, 600, ept % 600, d_hid, True)

    acc1 = a0[:n] + a1[:n]
    g2 = _tc_call(
        functools.partial(_tc2_body, n),
        jax.ShapeDtypeStruct((n, d_out), jnp.float32),
        acc1, g1, dp0, dp1,
        gamma1.reshape(1, d_hid), beta1.reshape(1, d_hid), W2,
    )

    b0, b1_ = _prop_call(g2, ei, zeros_nd[:, :d_out],
                         N_PAD, ept, ept // 1000, 1000, ept % 1000, d_out,
                         False)

    acc2 = b0[:n] + b1_[:n]
    out = _tc_call(
        functools.partial(_tc3_body, n),
        jax.ShapeDtypeStruct((n, d_out), jnp.float32),
        acc2, g2, dp0, dp1, b2.reshape(1, d_out),
    )
    return out
